# baseline (speedup 1.0000x reference)
import functools

import jax
import jax.numpy as jnp
from jax import lax
from jax.experimental import pallas as pl
from jax.experimental.pallas import tpu as pltpu
from jax.experimental.pallas import tpu_sc as plsc

N = 10000
E = 320000
D = 128
NSLOPE = 0.2
C = 48
NCORES = 2
NSUB = 16
NW = NCORES * NSUB
EPW = E // NW
CH = 208
TAIL_OFF = CH * C
NP = CH // 2


def _wh_body(x_ref, wt_ref, b_ref, o_ref):
    o_ref[...] = (
        jnp.dot(x_ref[...], wt_ref[...], preferred_element_type=jnp.float32)
        + b_ref[...]
    )


def _wh_matmul(x, wt, b2):
    return pl.pallas_call(
        _wh_body,
        out_shape=jax.ShapeDtypeStruct((N, D), jnp.float32),
    )(x, wt, b2)


def _edge_body(wh, srcl, dstl, a128, zeros, out, dout,
               gsrc0, gsrc1, gdst0, gdst1, cdst0, cdst1, sdst0, sdst1,
               whs0, whs1, whd0, whd1, roww0, roww1, tsrc, tdst,
               ebuf, tbuf, av, denom_v, acc,
               ga0, gb0, ga1, gb1, is0, is1, cs0, cs1, zs0, zs1,
               ss0, ss1):
    gsem = ((ga0, gb0), (ga1, gb1))
    isem = (is0, is1)
    csem = (cs0, cs1)
    zsem = (zs0, zs1)
    ssem = (ss0, ss1)
    gsrc = [gsrc0, gsrc1]
    gdst = [gdst0, gdst1]
    cdst = [cdst0, cdst1]
    sdst = [sdst0, sdst1]
    whs = [whs0, whs1]
    whd = [whd0, whd1]
    roww = [roww0, roww1]
    cid = lax.axis_index("c")
    sid = lax.axis_index("s")

    pltpu.sync_copy(a128, av)

    @pl.when(sid == 0)
    def _init():
        pltpu.sync_copy(zeros, acc)

    z16 = jnp.zeros((16,), jnp.float32)

    def zden(t, c2):
        denom_v[pl.ds(t * 16, 16)] = z16
        return c2

    lax.fori_loop(0, N // 16, zden, 0)
    plsc.subcore_barrier()

    a_regs = [av[pl.ds(16 * k, 16)] for k in range(8)]
    iota = lax.iota(jnp.int32, 16)
    row16 = iota * 16
    base_w = (cid * NSUB + sid) * EPW

    def esl(c):
        return pl.ds(base_w + c * C, C)

    def issue_gidx(c, s):
        pltpu.async_copy(srcl.at[esl(c)], gsrc[s], isem[s])
        pltpu.async_copy(dstl.at[esl(c)], gdst[s], isem[s])

    def wait_gidx(s):
        pltpu.make_async_copy(srcl.at[esl(0)], gsrc[s], isem[s]).wait()
        pltpu.make_async_copy(dstl.at[esl(0)], gdst[s], isem[s]).wait()

    def issue_cdst(c, s):
        pltpu.async_copy(dstl.at[esl(c)], cdst[s], csem[s])

    def wait_cdst(s):
        pltpu.make_async_copy(dstl.at[esl(0)], cdst[s], csem[s]).wait()

    def issue_sdst(c, s):
        pltpu.async_copy(dstl.at[esl(c)], sdst[s], zsem[s])

    def wait_sdst(s):
        pltpu.make_async_copy(dstl.at[esl(0)], sdst[s], zsem[s]).wait()

    def issue_gathers(s):
        pltpu.async_copy(wh.at[gsrc[s]], whs[s], gsem[s][0])
        pltpu.async_copy(wh.at[gdst[s]], whd[s], gsem[s][1])

    def wait_gathers(s):
        pltpu.make_async_copy(wh.at[gsrc[s]], whs[s], gsem[s][0]).wait()
        pltpu.make_async_copy(wh.at[gdst[s]], whd[s], gsem[s][1]).wait()

    def issue_scatter(s):
        pltpu.async_copy(roww[s], acc.at[sdst[s]], ssem[s], add=True)

    def drain_scatter(s):
        pltpu.make_async_copy(roww[s], acc.at[sdst[s]], ssem[s]).wait()

    def compute_on(whs_s, whd_s, roww_s, dst_s, ng):
        def score(g, c2):
            gb = g * 16
            for i16 in range(16):
                r = gb + i16

                def term(k):
                    sl = pl.ds(16 * k, 16)
                    sv = whs_s[r, sl] + whd_s[r, sl]
                    return a_regs[k] * jnp.maximum(sv, NSLOPE * sv)

                acc0 = ((term(0) + term(2)) + (term(4) + term(6)))
                acc1 = ((term(1) + term(3)) + (term(5) + term(7)))
                tbuf[pl.ds(i16 * 16, 16)] = acc0 + acc1
            cols = [plsc.load_gather(tbuf, [row16 + k]) for k in range(16)]
            while len(cols) > 1:
                cols = [cols[i] + cols[i + 1] for i in range(0, len(cols), 2)]
            w16 = jnp.exp(cols[0])
            ebuf[pl.ds(gb, 16)] = w16
            dst16 = dst_s[pl.ds(gb, 16)]
            plsc.addupdate_scatter(denom_v, [dst16], w16)
            return c2

        lax.fori_loop(0, ng, score, 0)

        def weight(i, c2):
            wv = plsc.load_gather(ebuf, [jnp.full((16,), i, jnp.int32)])
            for k in range(8):
                sl = pl.ds(16 * k, 16)
                roww_s[i, sl] = whs_s[i, sl] * wv
            return c2

        lax.fori_loop(0, ng * 16, weight, 0)

    def compute(s):
        compute_on(whs[s], whd[s], roww[s], cdst[s], C // 16)

    issue_gidx(0, 0)
    issue_gidx(1, 1)
    issue_cdst(0, 0)
    issue_cdst(1, 1)
    wait_gidx(0)
    wait_gidx(1)
    issue_gathers(0)
    issue_gathers(1)

    def slot_step(j, s, c):
        wait_gathers(s)

        @pl.when(c + 2 < CH)
        def _pg():
            issue_gidx(c + 2, s)

        @pl.when(j >= 1)
        def _dr():
            drain_scatter(s)

        issue_sdst(c, s)
        wait_cdst(s)
        compute(s)

        @pl.when(c + 2 < CH)
        def _pc():
            issue_cdst(c + 2, s)

        wait_sdst(s)
        issue_scatter(s)

        @pl.when(c + 2 < CH)
        def _ng():
            wait_gidx(s)
            issue_gathers(s)

    def pair(j, carry):
        c0 = 2 * j
        slot_step(j, 0, c0)
        slot_step(j, 1, c0 + 1)
        return carry

    lax.fori_loop(0, NP, pair, 0)
    drain_scatter(0)
    drain_scatter(1)

    pltpu.sync_copy(srcl.at[pl.ds(base_w + TAIL_OFF, 16)], tsrc)
    pltpu.sync_copy(dstl.at[pl.ds(base_w + TAIL_OFF, 16)], tdst)
    whs_t = whs0.at[pl.ds(0, 16)]
    whd_t = whd0.at[pl.ds(0, 16)]
    roww_t = roww0.at[pl.ds(0, 16)]
    g1 = pltpu.async_copy(wh.at[tsrc], whs_t, gsem[0][0])
    g2 = pltpu.async_copy(wh.at[tdst], whd_t, gsem[0][1])
    g1.wait()
    g2.wait()
    compute_on(whs0, whd0, roww0, tdst, 1)
    pltpu.async_copy(roww_t, acc.at[tdst], ssem[0], add=True).wait()

    pltpu.sync_copy(denom_v, dout.at[cid * NSUB + sid])
    plsc.subcore_barrier()

    @pl.when(sid == 0)
    def _flush():
        pltpu.sync_copy(acc, out.at[cid])


_IDX = pltpu.VMEM((C,), jnp.int32)
_ROWS = pltpu.VMEM((C, D), jnp.float32)
_edge_pass = functools.partial(
    pl.kernel,
    out_type=(
        jax.ShapeDtypeStruct((NCORES, N, D), jnp.float32),
        jax.ShapeDtypeStruct((NW, N), jnp.float32),
    ),
    mesh=plsc.VectorSubcoreMesh(core_axis_name="c", subcore_axis_name="s"),
    compiler_params=pltpu.CompilerParams(needs_layout_passes=False),
    scratch_types=(
        [_IDX] * 8
        + [_ROWS] * 6
        + [
            pltpu.VMEM((16,), jnp.int32),
            pltpu.VMEM((16,), jnp.int32),
            pltpu.VMEM((C,), jnp.float32),
            pltpu.VMEM((256,), jnp.float32),
            pltpu.VMEM((D,), jnp.float32),
            pltpu.VMEM((N,), jnp.float32),
            pltpu.VMEM_SHARED((N, D), jnp.float32),
        ]
        + [pltpu.SemaphoreType.DMA] * 12
    ),
)(_edge_body)


def _fin_body(p_ref, d_ref, o_ref):
    num = p_ref[0] + p_ref[1]
    den = jnp.sum(d_ref[...], axis=0)
    o_ref[...] = jax.nn.sigmoid(num / (den[:, None] + 1e-16))


def _finalize(parts, dens):
    return pl.pallas_call(
        _fin_body,
        out_shape=jax.ShapeDtypeStruct((N, D), jnp.float32),
    )(parts, dens)


def kernel(x, edge_index, W, b, a):
    wh = _wh_matmul(x, W.T, b[None, :])
    src = edge_index[0]
    dst = edge_index[1]
    zeros = jnp.zeros((N, D), jnp.float32)
    parts, dens = _edge_pass(wh, src, dst, a, zeros)
    return _finalize(parts, dens)


# speedup vs baseline: 1.5159x; 1.5159x over previous
import functools

import jax
import jax.numpy as jnp
from jax import lax
from jax.experimental import pallas as pl
from jax.experimental.pallas import tpu as pltpu
from jax.experimental.pallas import tpu_sc as plsc

N = 10000
E = 320000
D = 128
NSLOPE = 0.2
C = 48
NCORES = 2
NSUB = 16
NW = NCORES * NSUB
EPW = E // NW
CH = 208
TAIL_OFF = CH * C
NP = CH // 2


def _wh_body(x_ref, wt_ref, b_ref, o_ref):
    o_ref[...] = (
        jnp.dot(x_ref[...], wt_ref[...], preferred_element_type=jnp.float32)
        + b_ref[...]
    )


def _wh_matmul(x, wt, b2):
    return pl.pallas_call(
        _wh_body,
        out_shape=jax.ShapeDtypeStruct((N, D), jnp.float32),
    )(x, wt, b2)


def _edge_body(wh, srcl, dstl, a128, zeros, out, dout,
               gsrc0, gsrc1, gdst0, gdst1, cdst0, cdst1, sdst0, sdst1,
               whs0, whs1, whd0, whd1, roww0, roww1, tsrc, tdst,
               ebuf, tbuf, av, denom_v, acc,
               ga0, gb0, ga1, gb1, is0, is1, cs0, cs1, zs0, zs1,
               ss0, ss1):
    gsem = ((ga0, gb0), (ga1, gb1))
    isem = (is0, is1)
    csem = (cs0, cs1)
    zsem = (zs0, zs1)
    ssem = (ss0, ss1)
    gsrc = [gsrc0, gsrc1]
    gdst = [gdst0, gdst1]
    cdst = [cdst0, cdst1]
    sdst = [sdst0, sdst1]
    whs = [whs0, whs1]
    whd = [whd0, whd1]
    roww = [roww0, roww1]
    cid = lax.axis_index("c")
    sid = lax.axis_index("s")

    pltpu.sync_copy(a128, av)

    @pl.when(sid == 0)
    def _init():
        pltpu.sync_copy(zeros, acc)

    z16 = jnp.zeros((16,), jnp.float32)

    def zden(t, c2):
        denom_v[pl.ds(t * 16, 16)] = z16
        return c2

    lax.fori_loop(0, N // 16, zden, 0)
    plsc.subcore_barrier()

    a_regs = [av[pl.ds(16 * k, 16)] for k in range(8)]
    iota = lax.iota(jnp.int32, 16)
    row16 = iota * 16
    base_w = (cid * NSUB + sid) * EPW

    def esl(c):
        return pl.ds(base_w + c * C, C)

    def issue_gidx(c, s):
        pltpu.async_copy(srcl.at[esl(c)], gsrc[s], isem[s])
        pltpu.async_copy(dstl.at[esl(c)], gdst[s], isem[s])

    def wait_gidx(s):
        pltpu.make_async_copy(srcl.at[esl(0)], gsrc[s], isem[s]).wait()
        pltpu.make_async_copy(dstl.at[esl(0)], gdst[s], isem[s]).wait()

    def issue_cdst(c, s):
        pltpu.async_copy(dstl.at[esl(c)], cdst[s], csem[s])

    def wait_cdst(s):
        pltpu.make_async_copy(dstl.at[esl(0)], cdst[s], csem[s]).wait()

    def issue_sdst(c, s):
        pltpu.async_copy(dstl.at[esl(c)], sdst[s], zsem[s])

    def wait_sdst(s):
        pltpu.make_async_copy(dstl.at[esl(0)], sdst[s], zsem[s]).wait()

    def issue_gathers(s):
        pltpu.async_copy(wh.at[gsrc[s]], whs[s], gsem[s][0])
        pltpu.async_copy(wh.at[gdst[s]], whd[s], gsem[s][1])

    def wait_gathers(s):
        pltpu.make_async_copy(wh.at[gsrc[s]], whs[s], gsem[s][0]).wait()
        pltpu.make_async_copy(wh.at[gdst[s]], whd[s], gsem[s][1]).wait()

    def issue_scatter(s):
        pltpu.async_copy(roww[s], acc.at[sdst[s]], ssem[s], add=True)

    def drain_scatter(s):
        pltpu.make_async_copy(roww[s], acc.at[sdst[s]], ssem[s]).wait()

    def compute_on(whs_s, whd_s, roww_s, dst_s, ng):
        def score(g, c2):
            gb = g * 16
            for i16 in range(16):
                r = gb + i16

                def term(k):
                    sl = pl.ds(16 * k, 16)
                    sv = whs_s[r, sl] + whd_s[r, sl]
                    return a_regs[k] * jnp.maximum(sv, NSLOPE * sv)

                acc0 = ((term(0) + term(2)) + (term(4) + term(6)))
                acc1 = ((term(1) + term(3)) + (term(5) + term(7)))
                tbuf[pl.ds(i16 * 16, 16)] = acc0 + acc1
            cols = [plsc.load_gather(tbuf, [row16 + k]) for k in range(16)]
            while len(cols) > 1:
                cols = [cols[i] + cols[i + 1] for i in range(0, len(cols), 2)]
            w16 = jnp.exp(cols[0])
            ebuf[pl.ds(gb, 16)] = w16
            dst16 = dst_s[pl.ds(gb, 16)]
            plsc.addupdate_scatter(denom_v, [dst16], w16)
            return c2

        lax.fori_loop(0, ng, score, 0)

        @plsc.parallel_loop(0, ng * 16, 1, unroll=4)
        def weight(i):
            wv = plsc.load_gather(ebuf, [jnp.full((16,), i, jnp.int32)])
            for k in range(8):
                sl = pl.ds(16 * k, 16)
                roww_s[i, sl] = whs_s[i, sl] * wv

    def compute(s):
        compute_on(whs[s], whd[s], roww[s], cdst[s], C // 16)

    issue_gidx(0, 0)
    issue_gidx(1, 1)
    issue_cdst(0, 0)
    issue_cdst(1, 1)
    wait_gidx(0)
    wait_gidx(1)
    issue_gathers(0)
    issue_gathers(1)

    def slot_step(j, s, c):
        wait_gathers(s)

        @pl.when(c + 2 < CH)
        def _pg():
            issue_gidx(c + 2, s)

        @pl.when(j >= 1)
        def _dr():
            drain_scatter(s)

        issue_sdst(c, s)
        wait_cdst(s)
        compute(s)

        @pl.when(c + 2 < CH)
        def _pc():
            issue_cdst(c + 2, s)

        wait_sdst(s)
        issue_scatter(s)

        @pl.when(c + 2 < CH)
        def _ng():
            wait_gidx(s)
            issue_gathers(s)

    def pair(j, carry):
        c0 = 2 * j
        slot_step(j, 0, c0)
        slot_step(j, 1, c0 + 1)
        return carry

    lax.fori_loop(0, NP, pair, 0)
    drain_scatter(0)
    drain_scatter(1)

    pltpu.sync_copy(srcl.at[pl.ds(base_w + TAIL_OFF, 16)], tsrc)
    pltpu.sync_copy(dstl.at[pl.ds(base_w + TAIL_OFF, 16)], tdst)
    whs_t = whs0.at[pl.ds(0, 16)]
    whd_t = whd0.at[pl.ds(0, 16)]
    roww_t = roww0.at[pl.ds(0, 16)]
    g1 = pltpu.async_copy(wh.at[tsrc], whs_t, gsem[0][0])
    g2 = pltpu.async_copy(wh.at[tdst], whd_t, gsem[0][1])
    g1.wait()
    g2.wait()
    compute_on(whs0, whd0, roww0, tdst, 1)
    pltpu.async_copy(roww_t, acc.at[tdst], ssem[0], add=True).wait()

    pltpu.sync_copy(denom_v, dout.at[cid * NSUB + sid])
    plsc.subcore_barrier()

    @pl.when(sid == 0)
    def _flush():
        pltpu.sync_copy(acc, out.at[cid])


_IDX = pltpu.VMEM((C,), jnp.int32)
_ROWS = pltpu.VMEM((C, D), jnp.float32)
_edge_pass = functools.partial(
    pl.kernel,
    out_type=(
        jax.ShapeDtypeStruct((NCORES, N, D), jnp.float32),
        jax.ShapeDtypeStruct((NW, N), jnp.float32),
    ),
    mesh=plsc.VectorSubcoreMesh(core_axis_name="c", subcore_axis_name="s"),
    compiler_params=pltpu.CompilerParams(needs_layout_passes=False),
    scratch_types=(
        [_IDX] * 8
        + [_ROWS] * 6
        + [
            pltpu.VMEM((16,), jnp.int32),
            pltpu.VMEM((16,), jnp.int32),
            pltpu.VMEM((C,), jnp.float32),
            pltpu.VMEM((256,), jnp.float32),
            pltpu.VMEM((D,), jnp.float32),
            pltpu.VMEM((N,), jnp.float32),
            pltpu.VMEM_SHARED((N, D), jnp.float32),
        ]
        + [pltpu.SemaphoreType.DMA] * 12
    ),
)(_edge_body)


def _fin_body(p_ref, d_ref, o_ref):
    num = p_ref[0] + p_ref[1]
    den = jnp.sum(d_ref[...], axis=0)
    o_ref[...] = jax.nn.sigmoid(num / (den[:, None] + 1e-16))


def _finalize(parts, dens):
    return pl.pallas_call(
        _fin_body,
        out_shape=jax.ShapeDtypeStruct((N, D), jnp.float32),
    )(parts, dens)


def kernel(x, edge_index, W, b, a):
    wh = _wh_matmul(x, W.T, b[None, :])
    src = edge_index[0]
    dst = edge_index[1]
    zeros = jnp.zeros((N, D), jnp.float32)
    parts, dens = _edge_pass(wh, src, dst, a, zeros)
    return _finalize(parts, dens)


# speedup vs baseline: 1.5428x; 1.0177x over previous
import functools

import jax
import jax.numpy as jnp
from jax import lax
from jax.experimental import pallas as pl
from jax.experimental.pallas import tpu as pltpu
from jax.experimental.pallas import tpu_sc as plsc

N = 10000
E = 320000
D = 128
NSLOPE = 0.2
C = 48
NCORES = 2
NSUB = 16
NW = NCORES * NSUB
EPW = E // NW
CH = 208
TAIL_OFF = CH * C
NP = CH // 2


def _wh_body(x_ref, wt_ref, b_ref, o_ref):
    o_ref[...] = (
        jnp.dot(x_ref[...], wt_ref[...], preferred_element_type=jnp.float32)
        + b_ref[...]
    )


def _wh_matmul(x, wt, b2):
    return pl.pallas_call(
        _wh_body,
        out_shape=jax.ShapeDtypeStruct((N, D), jnp.float32),
    )(x, wt, b2)


def _edge_body(wh, srcl, dstl, a128, zeros, out, dout,
               gsrc0, gsrc1, gdst0, gdst1, cdst0, cdst1, sdst0, sdst1,
               whs0, whs1, whd0, whd1, roww0, roww1, tsrc, tdst,
               ebuf, tbuf, av, denom_v, acc,
               ga0, gb0, ga1, gb1, is0, is1, cs0, cs1, zs0, zs1,
               ss0, ss1):
    gsem = ((ga0, gb0), (ga1, gb1))
    isem = (is0, is1)
    csem = (cs0, cs1)
    zsem = (zs0, zs1)
    ssem = (ss0, ss1)
    gsrc = [gsrc0, gsrc1]
    gdst = [gdst0, gdst1]
    cdst = [cdst0, cdst1]
    sdst = [sdst0, sdst1]
    whs = [whs0, whs1]
    whd = [whd0, whd1]
    roww = [roww0, roww1]
    cid = lax.axis_index("c")
    sid = lax.axis_index("s")

    pltpu.sync_copy(a128, av)

    @pl.when(sid == 0)
    def _init():
        pltpu.sync_copy(zeros, acc)

    z16 = jnp.zeros((16,), jnp.float32)

    def zden(t, c2):
        denom_v[pl.ds(t * 16, 16)] = z16
        return c2

    lax.fori_loop(0, N // 16, zden, 0)
    plsc.subcore_barrier()

    a_regs = [av[pl.ds(16 * k, 16)] for k in range(8)]
    iota = lax.iota(jnp.int32, 16)
    row16 = iota * 16
    base_w = (cid * NSUB + sid) * EPW

    def esl(c):
        return pl.ds(base_w + c * C, C)

    def issue_gidx(c, s):
        pltpu.async_copy(srcl.at[esl(c)], gsrc[s], isem[s])
        pltpu.async_copy(dstl.at[esl(c)], gdst[s], isem[s])

    def wait_gidx(s):
        pltpu.make_async_copy(srcl.at[esl(0)], gsrc[s], isem[s]).wait()
        pltpu.make_async_copy(dstl.at[esl(0)], gdst[s], isem[s]).wait()

    def issue_cdst(c, s):
        pltpu.async_copy(dstl.at[esl(c)], cdst[s], csem[s])

    def wait_cdst(s):
        pltpu.make_async_copy(dstl.at[esl(0)], cdst[s], csem[s]).wait()

    def issue_sdst(c, s):
        pltpu.async_copy(dstl.at[esl(c)], sdst[s], zsem[s])

    def wait_sdst(s):
        pltpu.make_async_copy(dstl.at[esl(0)], sdst[s], zsem[s]).wait()

    def issue_gathers(s):
        pltpu.async_copy(wh.at[gsrc[s]], whs[s], gsem[s][0])
        pltpu.async_copy(wh.at[gdst[s]], whd[s], gsem[s][1])

    def wait_gathers(s):
        pltpu.make_async_copy(wh.at[gsrc[s]], whs[s], gsem[s][0]).wait()
        pltpu.make_async_copy(wh.at[gdst[s]], whd[s], gsem[s][1]).wait()

    def issue_scatter(s):
        pltpu.async_copy(roww[s], acc.at[sdst[s]], ssem[s], add=True)

    def drain_scatter(s):
        pltpu.make_async_copy(roww[s], acc.at[sdst[s]], ssem[s]).wait()

    def compute_on(whs_s, whd_s, roww_s, dst_s, ng):
        @plsc.parallel_loop(0, ng, 1, unroll=2)
        def score(g):
            gb = g * 16
            tb = g * 256
            for i16 in range(16):
                r = gb + i16

                def term(k):
                    sl = pl.ds(16 * k, 16)
                    sv = whs_s[r, sl] + whd_s[r, sl]
                    return a_regs[k] * jnp.maximum(sv, NSLOPE * sv)

                acc0 = ((term(0) + term(2)) + (term(4) + term(6)))
                acc1 = ((term(1) + term(3)) + (term(5) + term(7)))
                tbuf[pl.ds(tb + i16 * 16, 16)] = acc0 + acc1
            tbv = jnp.full((16,), tb, jnp.int32) + row16
            cols = [plsc.load_gather(tbuf, [tbv + k]) for k in range(16)]
            while len(cols) > 1:
                cols = [cols[i] + cols[i + 1] for i in range(0, len(cols), 2)]
            w16 = jnp.exp(cols[0])
            ebuf[pl.ds(gb, 16)] = w16
            dst16 = dst_s[pl.ds(gb, 16)]
            plsc.addupdate_scatter(denom_v, [dst16], w16)

        @plsc.parallel_loop(0, ng * 16, 1, unroll=4)
        def weight(i):
            wv = plsc.load_gather(ebuf, [jnp.full((16,), i, jnp.int32)])
            for k in range(8):
                sl = pl.ds(16 * k, 16)
                roww_s[i, sl] = whs_s[i, sl] * wv

    def compute(s):
        compute_on(whs[s], whd[s], roww[s], cdst[s], C // 16)

    issue_gidx(0, 0)
    issue_gidx(1, 1)
    issue_cdst(0, 0)
    issue_cdst(1, 1)
    wait_gidx(0)
    wait_gidx(1)
    issue_gathers(0)
    issue_gathers(1)

    def slot_step(j, s, c):
        wait_gathers(s)

        @pl.when(c + 2 < CH)
        def _pg():
            issue_gidx(c + 2, s)

        @pl.when(j >= 1)
        def _dr():
            drain_scatter(s)

        issue_sdst(c, s)
        wait_cdst(s)
        compute(s)

        @pl.when(c + 2 < CH)
        def _pc():
            issue_cdst(c + 2, s)

        wait_sdst(s)
        issue_scatter(s)

        @pl.when(c + 2 < CH)
        def _ng():
            wait_gidx(s)
            issue_gathers(s)

    def pair(j, carry):
        c0 = 2 * j
        slot_step(j, 0, c0)
        slot_step(j, 1, c0 + 1)
        return carry

    lax.fori_loop(0, NP, pair, 0)
    drain_scatter(0)
    drain_scatter(1)

    pltpu.sync_copy(srcl.at[pl.ds(base_w + TAIL_OFF, 16)], tsrc)
    pltpu.sync_copy(dstl.at[pl.ds(base_w + TAIL_OFF, 16)], tdst)
    whs_t = whs0.at[pl.ds(0, 16)]
    whd_t = whd0.at[pl.ds(0, 16)]
    roww_t = roww0.at[pl.ds(0, 16)]
    g1 = pltpu.async_copy(wh.at[tsrc], whs_t, gsem[0][0])
    g2 = pltpu.async_copy(wh.at[tdst], whd_t, gsem[0][1])
    g1.wait()
    g2.wait()
    compute_on(whs0, whd0, roww0, tdst, 1)
    pltpu.async_copy(roww_t, acc.at[tdst], ssem[0], add=True).wait()

    pltpu.sync_copy(denom_v, dout.at[cid * NSUB + sid])
    plsc.subcore_barrier()

    @pl.when(sid == 0)
    def _flush():
        pltpu.sync_copy(acc, out.at[cid])


_IDX = pltpu.VMEM((C,), jnp.int32)
_ROWS = pltpu.VMEM((C, D), jnp.float32)
_edge_pass = functools.partial(
    pl.kernel,
    out_type=(
        jax.ShapeDtypeStruct((NCORES, N, D), jnp.float32),
        jax.ShapeDtypeStruct((NW, N), jnp.float32),
    ),
    mesh=plsc.VectorSubcoreMesh(core_axis_name="c", subcore_axis_name="s"),
    compiler_params=pltpu.CompilerParams(needs_layout_passes=False),
    scratch_types=(
        [_IDX] * 8
        + [_ROWS] * 6
        + [
            pltpu.VMEM((16,), jnp.int32),
            pltpu.VMEM((16,), jnp.int32),
            pltpu.VMEM((C,), jnp.float32),
            pltpu.VMEM((768,), jnp.float32),
            pltpu.VMEM((D,), jnp.float32),
            pltpu.VMEM((N,), jnp.float32),
            pltpu.VMEM_SHARED((N, D), jnp.float32),
        ]
        + [pltpu.SemaphoreType.DMA] * 12
    ),
)(_edge_body)


def _fin_body(p_ref, d_ref, o_ref):
    num = p_ref[0] + p_ref[1]
    den = jnp.sum(d_ref[...], axis=0)
    o_ref[...] = jax.nn.sigmoid(num / (den[:, None] + 1e-16))


def _finalize(parts, dens):
    return pl.pallas_call(
        _fin_body,
        out_shape=jax.ShapeDtypeStruct((N, D), jnp.float32),
    )(parts, dens)


def kernel(x, edge_index, W, b, a):
    wh = _wh_matmul(x, W.T, b[None, :])
    src = edge_index[0]
    dst = edge_index[1]
    zeros = jnp.zeros((N, D), jnp.float32)
    parts, dens = _edge_pass(wh, src, dst, a, zeros)
    return _finalize(parts, dens)
